# baseline (speedup 1.0000x reference)
import jax
import jax.numpy as jnp
import numpy as np
from jax.experimental import pallas as pl

BS = 16
N_NODES = 15135
N_FEAT = 128
HID = 64
NUM_LAYERS = 3
HID_FC = 512
N_CLASSES = 2
N_EDGES = 484320
K_POOL = 7568


def _tail_body(xf_ref, wl1_ref, bl1_ref, wl2_ref, bl2_ref, out_ref):
    h = jnp.maximum(
        jnp.dot(xf_ref[...], wl1_ref[...], preferred_element_type=jnp.float32)
        + bl1_ref[...][None, :], 0.0)
    o = jnp.dot(h, wl2_ref[...], preferred_element_type=jnp.float32) + bl2_ref[...][None, :]
    m = jnp.max(o, axis=-1, keepdims=True)
    lse = jnp.log(jnp.sum(jnp.exp(o - m), axis=-1, keepdims=True)) + m
    out_ref[...] = o - lse


def _tail(xf, Wl1, bl1, Wl2, bl2):
    return pl.pallas_call(
        _tail_body,
        out_shape=jax.ShapeDtypeStruct((BS, N_CLASSES), jnp.float32),
    )(xf, Wl1, bl1, Wl2, bl2)


def kernel(x, batch, edge_index, W1, b1, W2, b2, W3, b3, p, Wfc, bfc, Wl1, bl1, Wl2, bl2):
    loops = jnp.arange(N_NODES, dtype=edge_index.dtype)
    src = jnp.concatenate([edge_index[0], loops])
    dst = jnp.concatenate([edge_index[1], loops])
    deg = jnp.zeros((N_NODES,), jnp.float32).at[dst].add(1.0)
    dinv = jnp.where(deg > 0, 1.0 / jnp.sqrt(deg), 0.0)
    norm = dinv[src] * dinv[dst]

    def gcn(h, W, b):
        hw = h @ W
        msg = hw[:, src, :] * norm[None, :, None]
        agg = jnp.zeros_like(hw).at[:, dst, :].add(msg)
        return agg + b

    h1 = jax.nn.relu(gcn(x, W1, b1))
    h2 = jax.nn.relu(gcn(h1, W2, b2))
    h3 = jax.nn.relu(gcn(h2, W3, b3))
    xc = jnp.stack([h1, h2, h3], axis=-1).reshape(BS, N_NODES, NUM_LAYERS * HID)

    score = jnp.tanh((xc @ p) / jnp.linalg.norm(p))
    perm = jnp.argsort(-score, axis=-1)[:, :K_POOL]
    sv = jnp.take_along_axis(score, perm, axis=1)
    xp = jnp.take_along_axis(xc, perm[:, :, None], axis=1) * sv[:, :, None]

    xf = (xp @ Wfc + bfc).reshape(BS, K_POOL)
    return _tail(xf, Wl1, bl1, Wl2, bl2)


# speedup vs baseline: 3.9404x; 3.9404x over previous
import functools

import jax
import jax.numpy as jnp
import numpy as np
from jax import lax
from jax.experimental import pallas as pl
from jax.experimental.pallas import tpu as pltpu
from jax.experimental.pallas import tpu_sc as plsc

BS = 16
N_NODES = 15135
N_FEAT = 128
HID = 64
NUM_LAYERS = 3
HID_FC = 512
N_CLASSES = 2
N_EDGES = 484320
K_POOL = 7568
E_TOT = N_EDGES + N_NODES

NB = 32
RPB = 480
NP = NB * RPB
RPC = 16 * RPB
CH = 128
NC_T = 132
CAP = NC_T * CH
ACCR = RPC + 8
NPAIR = BS // 2
W2 = 2 * HID

_mesh = plsc.VectorSubcoreMesh(core_axis_name="c", subcore_axis_name="s")


@functools.partial(
    pl.kernel,
    out_type=jax.ShapeDtypeStruct((NP,), jnp.float32),
    mesh=_mesh,
    scratch_types=[
        pltpu.VMEM((NC_T, CH), jnp.int32),
        pltpu.VMEM((CH,), jnp.float32),
        pltpu.VMEM((RPB,), jnp.float32),
        pltpu.VMEM_SHARED((ACCR,), jnp.float32),
    ],
)
def _deg_kernel(dstT, out, dstv, onesv, zv, acc):
    c = lax.axis_index("c")
    s = lax.axis_index("s")
    w = c * 16 + s
    pltpu.sync_copy(dstT.at[w], dstv)
    for k in range(CH // 16):
        onesv[pl.ds(k * 16, 16)] = jnp.ones((16,), jnp.float32)

    def zfill(i, carry):
        zv[pl.ds(i * 16, 16)] = jnp.zeros((16,), jnp.float32)
        return carry

    lax.fori_loop(0, RPB // 16, zfill, 0)
    pltpu.sync_copy(zv, acc.at[pl.ds(s * RPB, RPB)])

    def chunk(j, carry):
        pltpu.sync_copy(onesv, acc.at[dstv.at[j]], add=True)
        return carry

    lax.fori_loop(0, NC_T, chunk, 0)
    pltpu.sync_copy(acc.at[pl.ds(s * RPB, RPB)], zv)
    pltpu.sync_copy(zv, out.at[pl.ds(w * RPB, RPB)])


@functools.partial(
    pl.kernel,
    out_type=jax.ShapeDtypeStruct((NPAIR * NP, W2), jnp.float32),
    mesh=_mesh,
    scratch_types=[
        pltpu.VMEM((NC_T, CH), jnp.int32),
        pltpu.VMEM((NC_T, CH), jnp.int32),
        pltpu.VMEM((CH,), jnp.int32),
        pltpu.VMEM((CH, W2), jnp.float32),
        pltpu.VMEM((RPB // 4, W2), jnp.float32),
        pltpu.VMEM_SHARED((ACCR, W2), jnp.float32),
    ],
    compiler_params=pltpu.CompilerParams(use_tc_tiling_on_sc=False),
)
def _spmm_kernel(u_flat, srcT, dstT, out, srcv, dstv, soffv, gbuf, zbuf, acc):
    c = lax.axis_index("c")
    s = lax.axis_index("s")
    w = c * 16 + s
    q_rows = RPB // 4
    pltpu.sync_copy(srcT.at[w], srcv)
    pltpu.sync_copy(dstT.at[w], dstv)

    def zfill(i, carry):
        for k in range(W2 // 16):
            zbuf[i, pl.ds(k * 16, 16)] = jnp.zeros((16,), jnp.float32)
        return carry

    lax.fori_loop(0, q_rows, zfill, 0)

    def b_body(i, carry):
        boff = i * NP
        for q in range(4):
            pltpu.sync_copy(zbuf, acc.at[pl.ds(s * RPB + q * q_rows, q_rows)])

        def chunk(j, carry2):
            for k in range(CH // 16):
                soffv[pl.ds(k * 16, 16)] = srcv[j, pl.ds(k * 16, 16)] + boff
            pltpu.sync_copy(u_flat.at[soffv], gbuf)
            pltpu.sync_copy(gbuf, acc.at[dstv.at[j]], add=True)
            return carry2

        lax.fori_loop(0, NC_T, chunk, 0)
        pltpu.sync_copy(acc.at[pl.ds(s * RPB, RPB)],
                        out.at[pl.ds(boff + w * RPB, RPB)])
        return carry

    lax.fori_loop(0, NPAIR, b_body, 0)


def _tail_body(xf_ref, wl1_ref, bl1_ref, wl2_ref, bl2_ref, out_ref):
    h = jnp.maximum(
        jnp.dot(xf_ref[...], wl1_ref[...], preferred_element_type=jnp.float32)
        + bl1_ref[...][None, :], 0.0)
    o = jnp.dot(h, wl2_ref[...], preferred_element_type=jnp.float32) + bl2_ref[...][None, :]
    m = jnp.max(o, axis=-1, keepdims=True)
    lse = jnp.log(jnp.sum(jnp.exp(o - m), axis=-1, keepdims=True)) + m
    out_ref[...] = o - lse


def _tail(xf, Wl1, bl1, Wl2, bl2):
    return pl.pallas_call(
        _tail_body,
        out_shape=jax.ShapeDtypeStruct((BS, N_CLASSES), jnp.float32),
    )(xf, Wl1, bl1, Wl2, bl2)


def kernel(x, batch, edge_index, W1, b1, W2_, b2, W3, b3, p, Wfc, bfc, Wl1, bl1, Wl2, bl2):
    loops = jnp.arange(N_NODES, dtype=jnp.int32)
    src = jnp.concatenate([edge_index[0].astype(jnp.int32), loops])
    dst = jnp.concatenate([edge_index[1].astype(jnp.int32), loops])
    bucket = dst // RPB
    order = jnp.argsort(bucket, stable=True)
    sb = src[order]
    db = dst[order]
    bb = bucket[order]
    counts = jnp.bincount(bucket, length=NB).astype(jnp.int32)
    starts = jnp.concatenate(
        [jnp.zeros((1,), jnp.int32), jnp.cumsum(counts)[:-1].astype(jnp.int32)])
    r = jnp.arange(E_TOT, dtype=jnp.int32) - starts[bb]
    srcT = jnp.full((NB, CAP), N_NODES, jnp.int32).at[bb, r].set(sb)
    dstT = jnp.full((NB, CAP), RPC, jnp.int32).at[bb, r].set(db - (bb // 16) * RPC)
    srcT = srcT.reshape(NB, NC_T, CH)
    dstT = dstT.reshape(NB, NC_T, CH)

    deg = _deg_kernel(dstT)
    dinv = jnp.where(deg > 0, lax.rsqrt(deg), 0.0)[:N_NODES]

    def layer(h, W):
        hw = h @ W
        u = hw * dinv[None, :, None]
        u = jnp.pad(u, ((0, 0), (0, NP - N_NODES), (0, 0)))
        up = u.reshape(NPAIR, 2, NP, HID).transpose(0, 2, 1, 3).reshape(NPAIR * NP, W2)
        sagg = _spmm_kernel(up, srcT, dstT)
        agg = sagg.reshape(NPAIR, NP, 2, HID).transpose(0, 2, 1, 3).reshape(BS, NP, HID)
        return agg[:, :N_NODES, :] * dinv[None, :, None]

    h1 = jax.nn.relu(layer(x, W1) + b1)
    h2 = jax.nn.relu(layer(h1, W2_) + b2)
    h3 = jax.nn.relu(layer(h2, W3) + b3)
    xc = jnp.stack([h1, h2, h3], axis=-1).reshape(BS, N_NODES, NUM_LAYERS * HID)

    score = jnp.tanh((xc @ p) / jnp.linalg.norm(p))
    sv, perm = lax.top_k(score, K_POOL)
    t = (xc @ Wfc).reshape(BS, N_NODES)
    xf = jnp.take_along_axis(t, perm, axis=1) * sv + bfc

    return _tail(xf, Wl1, bl1, Wl2, bl2)
